# baseline (speedup 1.0000x reference)
import jax
import jax.numpy as jnp
from jax import lax
from jax.experimental import pallas as pl
from jax.experimental.pallas import tpu as pltpu
from jax.experimental.pallas import tpu_sc as plsc

N = 131072
C = 190
NC = 2
NS = 16
NW = NC * NS
L = 16
TILE = 128

KTC = 320
TPW = (N // TILE - KTC) // NW
RPW = TPW * TILE
SC_BASE = KTC * TILE
VPT = TILE // L
NBUF = 4
AHEAD = NBUF - 1

TC_COLS = KTC * TILE
TC_BLK = 4096
TC_GRID = TC_COLS // TC_BLK
TC_SUB = TC_BLK // 512


def _sc_body(lt_hbm, tgt_hbm, out_hbm, tgt_v, fidx_v, vals_v, acc_v, sem):
    c = lax.axis_index("c")
    s = lax.axis_index("s")
    wid = s * NC + c
    base = SC_BASE + wid * RPW

    pltpu.sync_copy(tgt_hbm.at[pl.ds(base, RPW)], tgt_v)

    iot = lax.iota(jnp.int32, L)

    def build(g):
        def one(o, _):
            j = g * VPT + o
            t = tgt_v[pl.ds(j * L, L)]
            fidx_v[pl.ds(j * L, L)] = jnp.where(t != -1, t, 0)
            return 0
        lax.fori_loop(0, VPT, one, 0)

    def fire(g):
        slot = lax.rem(g, NBUF)
        cb = pl.multiple_of(base + g * TILE, TILE)
        pltpu.async_copy(
            lt_hbm.at[fidx_v.at[pl.ds(g * TILE, TILE)], pl.ds(cb, TILE)],
            vals_v.at[slot],
            sem,
        )

    for g0 in range(AHEAD):
        build(g0)
        fire(g0)

    def tile_body(g, carry):
        acc_sq, acc_ct = carry

        @pl.when(g + AHEAD < TPW)
        def _():
            build(g + AHEAD)
            fire(g + AHEAD)

        slot = lax.rem(g, NBUF)
        cb = pl.multiple_of(base + g * TILE, TILE)
        pltpu.make_async_copy(
            lt_hbm.at[fidx_v.at[pl.ds(g * TILE, TILE)], pl.ds(cb, TILE)],
            vals_v.at[slot],
            sem,
        ).wait()

        slot_vec = jnp.full((L,), slot, jnp.int32)
        for o in range(VPT):
            diag = o * L + iot
            v = plsc.load_gather(vals_v, [slot_vec, diag, diag])
            t = tgt_v[pl.ds(g * TILE + o * L, L)]
            mf = jnp.where(t != -1, 1.0, 0.0).astype(jnp.float32)
            d = 1.0 - v
            acc_sq = acc_sq + d * d * mf
            acc_ct = acc_ct + mf
        return acc_sq, acc_ct

    acc_sq, acc_ct = lax.fori_loop(
        0, TPW, tile_body,
        (jnp.zeros((L,), jnp.float32), jnp.zeros((L,), jnp.float32)),
    )

    acc_v[0, :] = acc_sq
    acc_v[1, :] = acc_ct
    pltpu.sync_copy(acc_v.at[0], out_hbm.at[wid])
    pltpu.sync_copy(acc_v.at[1], out_hbm.at[NW + wid])


def _tc_body(lt_ref, tgt_ref, sq_ref, ct_ref):
    b = pl.program_id(0)

    @pl.when(b == 0)
    def _():
        sq_ref[...] = jnp.zeros_like(sq_ref)
        ct_ref[...] = jnp.zeros_like(ct_ref)

    x = lt_ref[...]
    iota2d = lax.broadcasted_iota(jnp.int32, (C, 512), 0)
    for sub in range(TC_SUB):
        t = tgt_ref[sub, :]
        xs = x[:, sub * 512:(sub + 1) * 512]
        m = iota2d == t[None, :]
        d = 1.0 - xs
        contrib = jnp.sum(jnp.where(m, d * d, 0.0), axis=0)
        sq_ref[sub, :] += contrib
        ct_ref[sub, :] += (t != -1).astype(jnp.float32)


def _finish_body(p_ref, tsq_ref, tct_ref, o_ref):
    p = p_ref[...]
    total_sq = jnp.sum(p[0:NW, :]) + jnp.sum(tsq_ref[...])
    total_ct = jnp.sum(p[NW:2 * NW, :]) + jnp.sum(tct_ref[...])
    o_ref[...] = jnp.full((1, 1), total_sq / total_ct, jnp.float32)


@jax.jit
def kernel(contrast_logits, contrast_target):
    lt = contrast_logits.T
    tgt = contrast_target.astype(jnp.int32)

    mesh = plsc.VectorSubcoreMesh(core_axis_name="c", subcore_axis_name="s")
    sc_partials = pl.kernel(
        _sc_body,
        out_type=jax.ShapeDtypeStruct((2 * NW, L), jnp.float32),
        mesh=mesh,
        compiler_params=pltpu.CompilerParams(needs_layout_passes=False),
        scratch_types=[
            pltpu.VMEM((RPW,), jnp.int32),
            pltpu.VMEM((RPW,), jnp.int32),
            pltpu.VMEM((NBUF, TILE, TILE), jnp.float32),
            pltpu.VMEM((2, L), jnp.float32),
            pltpu.SemaphoreType.DMA,
        ],
    )(lt, tgt)

    tgt2d = tgt.reshape(N // 512, 512)
    tc_sq, tc_ct = pl.pallas_call(
        _tc_body,
        grid=(TC_GRID,),
        in_specs=[
            pl.BlockSpec((C, TC_BLK), lambda b: (0, b)),
            pl.BlockSpec((TC_SUB, 512), lambda b: (b, 0)),
        ],
        out_specs=[
            pl.BlockSpec((TC_SUB, 512), lambda b: (0, 0)),
            pl.BlockSpec((TC_SUB, 512), lambda b: (0, 0)),
        ],
        out_shape=[
            jax.ShapeDtypeStruct((TC_SUB, 512), jnp.float32),
            jax.ShapeDtypeStruct((TC_SUB, 512), jnp.float32),
        ],
    )(lt, tgt2d)

    loss = pl.pallas_call(
        _finish_body,
        out_shape=jax.ShapeDtypeStruct((1, 1), jnp.float32),
    )(sc_partials, tc_sq, tc_ct)
    return loss[0, 0]


# speedup vs baseline: 1.0023x; 1.0023x over previous
import jax
import jax.numpy as jnp
from jax import lax
from jax.experimental import pallas as pl
from jax.experimental.pallas import tpu as pltpu
from jax.experimental.pallas import tpu_sc as plsc

N = 131072
C = 190
NC = 2
NS = 16
NW = NC * NS
L = 16
TILE = 128

KTC = 448
TPW = (N // TILE - KTC) // NW
RPW = TPW * TILE
SC_BASE = KTC * TILE
VPT = TILE // L
NBUF = 3
AHEAD = NBUF - 1

TC_COLS = KTC * TILE
TC_BLK = 4096
TC_GRID = TC_COLS // TC_BLK
TC_SUB = TC_BLK // 512


def _sc_body(lt_hbm, tgt_hbm, out_hbm, tgt_v, fidx_v, vals_v, acc_v, sem):
    c = lax.axis_index("c")
    s = lax.axis_index("s")
    wid = s * NC + c
    base = SC_BASE + wid * RPW

    pltpu.sync_copy(tgt_hbm.at[pl.ds(base, RPW)], tgt_v)

    iot = lax.iota(jnp.int32, L)

    def build(g):
        def one(o, _):
            j = g * VPT + o
            t = tgt_v[pl.ds(j * L, L)]
            fidx_v[pl.ds(j * L, L)] = jnp.where(t != -1, t, 0)
            return 0
        lax.fori_loop(0, VPT, one, 0)

    def fire(g):
        slot = lax.rem(g, NBUF)
        cb = pl.multiple_of(base + g * TILE, TILE)
        pltpu.async_copy(
            lt_hbm.at[fidx_v.at[pl.ds(g * TILE, TILE)], pl.ds(cb, TILE)],
            vals_v.at[slot],
            sem,
        )

    for g0 in range(AHEAD):
        build(g0)
        fire(g0)

    def tile_body(g, carry):
        acc_sq, acc_ct = carry

        @pl.when(g + AHEAD < TPW)
        def _():
            build(g + AHEAD)
            fire(g + AHEAD)

        slot = lax.rem(g, NBUF)
        cb = pl.multiple_of(base + g * TILE, TILE)
        pltpu.make_async_copy(
            lt_hbm.at[fidx_v.at[pl.ds(g * TILE, TILE)], pl.ds(cb, TILE)],
            vals_v.at[slot],
            sem,
        ).wait()

        slot_vec = jnp.full((L,), slot, jnp.int32)
        for o in range(VPT):
            diag = o * L + iot
            v = plsc.load_gather(vals_v, [slot_vec, diag, diag])
            t = tgt_v[pl.ds(g * TILE + o * L, L)]
            mf = jnp.where(t != -1, 1.0, 0.0).astype(jnp.float32)
            d = 1.0 - v
            acc_sq = acc_sq + d * d * mf
            acc_ct = acc_ct + mf
        return acc_sq, acc_ct

    acc_sq, acc_ct = lax.fori_loop(
        0, TPW, tile_body,
        (jnp.zeros((L,), jnp.float32), jnp.zeros((L,), jnp.float32)),
    )

    acc_v[0, :] = acc_sq
    acc_v[1, :] = acc_ct
    pltpu.sync_copy(acc_v.at[0], out_hbm.at[wid])
    pltpu.sync_copy(acc_v.at[1], out_hbm.at[NW + wid])


def _tc_body(lt_ref, tgt_ref, sq_ref, ct_ref):
    b = pl.program_id(0)

    @pl.when(b == 0)
    def _():
        sq_ref[...] = jnp.zeros_like(sq_ref)
        ct_ref[...] = jnp.zeros_like(ct_ref)

    x = lt_ref[...]
    iota2d = lax.broadcasted_iota(jnp.int32, (C, 512), 0)
    for sub in range(TC_SUB):
        t = tgt_ref[sub, :]
        xs = x[:, sub * 512:(sub + 1) * 512]
        m = iota2d == t[None, :]
        sel = jnp.sum(jnp.where(m, xs, 0.0), axis=0)
        valid = (t != -1).astype(jnp.float32)
        d = 1.0 - sel
        sq_ref[sub, :] += valid * d * d
        ct_ref[sub, :] += valid


def _finish_body(p_ref, tsq_ref, tct_ref, o_ref):
    p = p_ref[...]
    total_sq = jnp.sum(p[0:NW, :]) + jnp.sum(tsq_ref[...])
    total_ct = jnp.sum(p[NW:2 * NW, :]) + jnp.sum(tct_ref[...])
    o_ref[...] = jnp.full((1, 1), total_sq / total_ct, jnp.float32)


@jax.jit
def kernel(contrast_logits, contrast_target):
    lt = contrast_logits.T
    tgt = contrast_target.astype(jnp.int32)

    mesh = plsc.VectorSubcoreMesh(core_axis_name="c", subcore_axis_name="s")
    sc_partials = pl.kernel(
        _sc_body,
        out_type=jax.ShapeDtypeStruct((2 * NW, L), jnp.float32),
        mesh=mesh,
        compiler_params=pltpu.CompilerParams(needs_layout_passes=False),
        scratch_types=[
            pltpu.VMEM((RPW,), jnp.int32),
            pltpu.VMEM((RPW,), jnp.int32),
            pltpu.VMEM((NBUF, TILE, TILE), jnp.float32),
            pltpu.VMEM((2, L), jnp.float32),
            pltpu.SemaphoreType.DMA,
        ],
    )(lt, tgt)

    tgt2d = tgt.reshape(N // 512, 512)
    tc_sq, tc_ct = pl.pallas_call(
        _tc_body,
        grid=(TC_GRID,),
        in_specs=[
            pl.BlockSpec((C, TC_BLK), lambda b: (0, b)),
            pl.BlockSpec((TC_SUB, 512), lambda b: (b, 0)),
        ],
        out_specs=[
            pl.BlockSpec((TC_SUB, 512), lambda b: (0, 0)),
            pl.BlockSpec((TC_SUB, 512), lambda b: (0, 0)),
        ],
        out_shape=[
            jax.ShapeDtypeStruct((TC_SUB, 512), jnp.float32),
            jax.ShapeDtypeStruct((TC_SUB, 512), jnp.float32),
        ],
    )(lt, tgt2d)

    loss = pl.pallas_call(
        _finish_body,
        out_shape=jax.ShapeDtypeStruct((1, 1), jnp.float32),
    )(sc_partials, tc_sq, tc_ct)
    return loss[0, 0]


# speedup vs baseline: 1.0150x; 1.0127x over previous
import jax
import jax.numpy as jnp
from jax import lax
from jax.experimental import pallas as pl
from jax.experimental.pallas import tpu as pltpu
from jax.experimental.pallas import tpu_sc as plsc

N = 131072
C = 190
NC = 2
NS = 16
NW = NC * NS
L = 16
TILE = 128

KTC = 320
TPW = (N // TILE - KTC) // NW
RPW = TPW * TILE
SC_BASE = KTC * TILE
VPT = TILE // L
NBUF = 3
AHEAD = NBUF - 1

TC_COLS = KTC * TILE
TC_BLK = 4096
TC_GRID = TC_COLS // TC_BLK
TC_SUB = TC_BLK // 512


def _sc_body(lt_hbm, tgt_hbm, out_hbm, tgt_v, fidx_v, vals_v, acc_v, sem):
    c = lax.axis_index("c")
    s = lax.axis_index("s")
    wid = s * NC + c
    base = SC_BASE + wid * RPW

    pltpu.sync_copy(tgt_hbm.at[pl.ds(base, RPW)], tgt_v)

    iot = lax.iota(jnp.int32, L)

    def build(g):
        def one(o, _):
            j = g * VPT + o
            t = tgt_v[pl.ds(j * L, L)]
            fidx_v[pl.ds(j * L, L)] = jnp.where(t != -1, t, 0)
            return 0
        lax.fori_loop(0, VPT, one, 0)

    def fire(g):
        slot = lax.rem(g, NBUF)
        cb = pl.multiple_of(base + g * TILE, TILE)
        pltpu.async_copy(
            lt_hbm.at[fidx_v.at[pl.ds(g * TILE, TILE)], pl.ds(cb, TILE)],
            vals_v.at[slot],
            sem,
        )

    for g0 in range(AHEAD):
        build(g0)
        fire(g0)

    def tile_body(g, carry):
        acc_sq, acc_ct = carry

        @pl.when(g + AHEAD < TPW)
        def _():
            build(g + AHEAD)
            fire(g + AHEAD)

        slot = lax.rem(g, NBUF)
        cb = pl.multiple_of(base + g * TILE, TILE)
        pltpu.make_async_copy(
            lt_hbm.at[fidx_v.at[pl.ds(g * TILE, TILE)], pl.ds(cb, TILE)],
            vals_v.at[slot],
            sem,
        ).wait()

        slot_vec = jnp.full((L,), slot, jnp.int32)
        for o in range(VPT):
            diag = o * L + iot
            v = plsc.load_gather(vals_v, [slot_vec, diag, diag])
            t = tgt_v[pl.ds(g * TILE + o * L, L)]
            mf = jnp.where(t != -1, 1.0, 0.0).astype(jnp.float32)
            d = 1.0 - v
            acc_sq = acc_sq + d * d * mf
            acc_ct = acc_ct + mf
        return acc_sq, acc_ct

    acc_sq, acc_ct = lax.fori_loop(
        0, TPW, tile_body,
        (jnp.zeros((L,), jnp.float32), jnp.zeros((L,), jnp.float32)),
    )

    acc_v[0, :] = acc_sq
    acc_v[1, :] = acc_ct
    pltpu.sync_copy(acc_v.at[0], out_hbm.at[wid])
    pltpu.sync_copy(acc_v.at[1], out_hbm.at[NW + wid])


def _tc_body(lt_ref, tgt_ref, sq_ref, ct_ref):
    b = pl.program_id(0)

    @pl.when(b == 0)
    def _():
        sq_ref[...] = jnp.zeros_like(sq_ref)
        ct_ref[...] = jnp.zeros_like(ct_ref)

    x = lt_ref[...]
    iota2d = lax.broadcasted_iota(jnp.int32, (C, 512), 0)
    for sub in range(TC_SUB):
        t = tgt_ref[sub, :]
        xs = x[:, sub * 512:(sub + 1) * 512]
        valid = (t != -1)
        m = (iota2d == t[None, :]) & valid[None, :]
        d = 1.0 - xs
        contrib = jnp.sum(jnp.where(m, d * d, 0.0), axis=0)
        sq_ref[sub, :] += contrib
        ct_ref[sub, :] += valid.astype(jnp.float32)


def _finish_body(p_ref, tsq_ref, tct_ref, o_ref):
    p = p_ref[...]
    total_sq = jnp.sum(p[0:NW, :]) + jnp.sum(tsq_ref[...])
    total_ct = jnp.sum(p[NW:2 * NW, :]) + jnp.sum(tct_ref[...])
    o_ref[...] = jnp.full((1, 1), total_sq / total_ct, jnp.float32)


@jax.jit
def kernel(contrast_logits, contrast_target):
    lt = contrast_logits.T
    tgt = contrast_target.astype(jnp.int32)

    mesh = plsc.VectorSubcoreMesh(core_axis_name="c", subcore_axis_name="s")
    sc_partials = pl.kernel(
        _sc_body,
        out_type=jax.ShapeDtypeStruct((2 * NW, L), jnp.float32),
        mesh=mesh,
        compiler_params=pltpu.CompilerParams(needs_layout_passes=False),
        scratch_types=[
            pltpu.VMEM((RPW,), jnp.int32),
            pltpu.VMEM((RPW,), jnp.int32),
            pltpu.VMEM((NBUF, TILE, TILE), jnp.float32),
            pltpu.VMEM((2, L), jnp.float32),
            pltpu.SemaphoreType.DMA,
        ],
    )(lt, tgt)

    tgt2d = tgt.reshape(N // 512, 512)
    tc_sq, tc_ct = pl.pallas_call(
        _tc_body,
        grid=(TC_GRID,),
        in_specs=[
            pl.BlockSpec((C, TC_BLK), lambda b: (0, b)),
            pl.BlockSpec((TC_SUB, 512), lambda b: (b, 0)),
        ],
        out_specs=[
            pl.BlockSpec((TC_SUB, 512), lambda b: (0, 0)),
            pl.BlockSpec((TC_SUB, 512), lambda b: (0, 0)),
        ],
        out_shape=[
            jax.ShapeDtypeStruct((TC_SUB, 512), jnp.float32),
            jax.ShapeDtypeStruct((TC_SUB, 512), jnp.float32),
        ],
    )(lt, tgt2d)

    loss = pl.pallas_call(
        _finish_body,
        out_shape=jax.ShapeDtypeStruct((1, 1), jnp.float32),
    )(sc_partials, tc_sq, tc_ct)
    return loss[0, 0]


# speedup vs baseline: 1.0393x; 1.0239x over previous
import jax
import jax.numpy as jnp
from jax import lax
from jax.experimental import pallas as pl
from jax.experimental.pallas import tpu as pltpu
from jax.experimental.pallas import tpu_sc as plsc

N = 131072
C = 190
NC = 2
NS = 16
NW = NC * NS
L = 16
TILE = 128

KTC = 128
TPW = (N // TILE - KTC) // NW
RPW = TPW * TILE
SC_BASE = KTC * TILE
VPT = TILE // L
NBUF = 3
AHEAD = NBUF - 1

TC_COLS = KTC * TILE
TC_BLK = 4096
TC_GRID = TC_COLS // TC_BLK
TC_SUB = TC_BLK // 512


def _sc_body(lt_hbm, tgt_hbm, out_hbm, tgt_v, fidx_v, vals_v, acc_v, sem):
    c = lax.axis_index("c")
    s = lax.axis_index("s")
    wid = s * NC + c
    base = SC_BASE + wid * RPW

    pltpu.sync_copy(tgt_hbm.at[pl.ds(base, RPW)], tgt_v)

    iot = lax.iota(jnp.int32, L)

    def build(g):
        def one(o, _):
            j = g * VPT + o
            t = tgt_v[pl.ds(j * L, L)]
            fidx_v[pl.ds(j * L, L)] = jnp.where(t != -1, t, 0)
            return 0
        lax.fori_loop(0, VPT, one, 0)

    def fire(g):
        slot = lax.rem(g, NBUF)
        cb = pl.multiple_of(base + g * TILE, TILE)
        pltpu.async_copy(
            lt_hbm.at[fidx_v.at[pl.ds(g * TILE, TILE)], pl.ds(cb, TILE)],
            vals_v.at[slot],
            sem,
        )

    for g0 in range(AHEAD):
        build(g0)
        fire(g0)

    def tile_body(g, carry):
        acc_sq, acc_ct = carry

        @pl.when(g + AHEAD < TPW)
        def _():
            build(g + AHEAD)
            fire(g + AHEAD)

        slot = lax.rem(g, NBUF)
        cb = pl.multiple_of(base + g * TILE, TILE)
        pltpu.make_async_copy(
            lt_hbm.at[fidx_v.at[pl.ds(g * TILE, TILE)], pl.ds(cb, TILE)],
            vals_v.at[slot],
            sem,
        ).wait()

        slot_vec = jnp.full((L,), slot, jnp.int32)
        for o in range(VPT):
            diag = o * L + iot
            v = plsc.load_gather(vals_v, [slot_vec, diag, diag])
            t = tgt_v[pl.ds(g * TILE + o * L, L)]
            mf = jnp.where(t != -1, 1.0, 0.0).astype(jnp.float32)
            d = 1.0 - v
            acc_sq = acc_sq + d * d * mf
            acc_ct = acc_ct + mf
        return acc_sq, acc_ct

    acc_sq, acc_ct = lax.fori_loop(
        0, TPW, tile_body,
        (jnp.zeros((L,), jnp.float32), jnp.zeros((L,), jnp.float32)),
    )

    acc_v[0, :] = acc_sq
    acc_v[1, :] = acc_ct
    pltpu.sync_copy(acc_v.at[0], out_hbm.at[wid])
    pltpu.sync_copy(acc_v.at[1], out_hbm.at[NW + wid])


def _tc_body(lt_ref, tgt_ref, sq_ref, ct_ref):
    b = pl.program_id(0)

    @pl.when(b == 0)
    def _():
        sq_ref[...] = jnp.zeros_like(sq_ref)
        ct_ref[...] = jnp.zeros_like(ct_ref)

    x = lt_ref[...]
    iota2d = lax.broadcasted_iota(jnp.int32, (C, 512), 0)
    for sub in range(TC_SUB):
        t = tgt_ref[sub, :]
        xs = x[:, sub * 512:(sub + 1) * 512]
        valid = (t != -1)
        m = (iota2d == t[None, :]) & valid[None, :]
        d = 1.0 - xs
        contrib = jnp.sum(jnp.where(m, d * d, 0.0), axis=0)
        sq_ref[sub, :] += contrib
        ct_ref[sub, :] += valid.astype(jnp.float32)


def _finish_body(p_ref, tsq_ref, tct_ref, o_ref):
    p = p_ref[...]
    total_sq = jnp.sum(p[0:NW, :]) + jnp.sum(tsq_ref[...])
    total_ct = jnp.sum(p[NW:2 * NW, :]) + jnp.sum(tct_ref[...])
    o_ref[...] = jnp.full((1, 1), total_sq / total_ct, jnp.float32)


@jax.jit
def kernel(contrast_logits, contrast_target):
    lt = contrast_logits.T
    tgt = contrast_target.astype(jnp.int32)

    mesh = plsc.VectorSubcoreMesh(core_axis_name="c", subcore_axis_name="s")
    sc_partials = pl.kernel(
        _sc_body,
        out_type=jax.ShapeDtypeStruct((2 * NW, L), jnp.float32),
        mesh=mesh,
        compiler_params=pltpu.CompilerParams(needs_layout_passes=False),
        scratch_types=[
            pltpu.VMEM((RPW,), jnp.int32),
            pltpu.VMEM((RPW,), jnp.int32),
            pltpu.VMEM((NBUF, TILE, TILE), jnp.float32),
            pltpu.VMEM((2, L), jnp.float32),
            pltpu.SemaphoreType.DMA,
        ],
    )(lt, tgt)

    tgt2d = tgt.reshape(N // 512, 512)
    tc_sq, tc_ct = pl.pallas_call(
        _tc_body,
        grid=(TC_GRID,),
        in_specs=[
            pl.BlockSpec((C, TC_BLK), lambda b: (0, b)),
            pl.BlockSpec((TC_SUB, 512), lambda b: (b, 0)),
        ],
        out_specs=[
            pl.BlockSpec((TC_SUB, 512), lambda b: (0, 0)),
            pl.BlockSpec((TC_SUB, 512), lambda b: (0, 0)),
        ],
        out_shape=[
            jax.ShapeDtypeStruct((TC_SUB, 512), jnp.float32),
            jax.ShapeDtypeStruct((TC_SUB, 512), jnp.float32),
        ],
    )(lt, tgt2d)

    loss = pl.pallas_call(
        _finish_body,
        out_shape=jax.ShapeDtypeStruct((1, 1), jnp.float32),
    )(sc_partials, tc_sq, tc_ct)
    return loss[0, 0]
